# baseline (speedup 1.0000x reference)
import functools

import jax
import jax.numpy as jnp
from jax import lax
from jax.experimental import pallas as pl
from jax.experimental.pallas import tpu as pltpu
from jax.experimental.pallas import tpu_sc as plsc

N = 10000
E = 320000
DF = 128
DE = 16
H = 128

NC = 2
NS = 16
NW = NC * NS
EPW = E // NW
CHUNK = 128
NPASS = 2
GCHUNK = 40
NCHUNK = NPASS * GCHUNK
EPAD = NCHUNK * CHUNK - EPW
RPW = 320
NROWS = NW * RPW

def _zero_share(zbuf, acc, base):
    pltpu.sync_copy(zbuf, acc.at[pl.ds(base, CHUNK)])
    pltpu.sync_copy(zbuf, acc.at[pl.ds(base + CHUNK, CHUNK)])
    pltpu.sync_copy(zbuf.at[pl.ds(0, RPW - 2 * CHUNK)],
                    acc.at[pl.ds(base + 2 * CHUNK, RPW - 2 * CHUNK)])


def _read_share(acc, buf, out_hbm, cid, base):
    for lo, n in ((0, CHUNK), (CHUNK, CHUNK), (2 * CHUNK, RPW - 2 * CHUNK)):
        pltpu.sync_copy(acc.at[pl.ds(base + lo, n)], buf.at[pl.ds(0, n)])
        pltpu.sync_copy(buf.at[pl.ds(0, n)], out_hbm.at[cid, pl.ds(base + lo, n)])


def _sc_degree_body(dsts, ones, zeros, out, didx, obuf, zbuf, acc):
    cid = lax.axis_index("c")
    sid = lax.axis_index("s")
    w = cid * NS + sid
    pltpu.sync_copy(dsts.at[w], didx)
    pltpu.sync_copy(ones, obuf)
    pltpu.sync_copy(zeros, zbuf)
    base = sid * RPW
    _zero_share(zbuf, acc, base)
    plsc.subcore_barrier()

    @pl.loop(0, NCHUNK)
    def _(j):
        pltpu.sync_copy(obuf, acc.at[didx.at[j]], add=True)

    plsc.subcore_barrier()
    _read_share(acc, zbuf, out, cid, base)


def _sc_scatter_body(g, srcs, dsts, zeros, out, sidx, didx, rows0, rows1,
                     acc, semg0, semg1):
    cid = lax.axis_index("c")
    sid = lax.axis_index("s")
    w = cid * NS + sid
    base = sid * RPW
    pltpu.sync_copy(zeros, rows0)
    _zero_share(rows0, acc, base)
    plsc.subcore_barrier()

    @pl.loop(0, NPASS)
    def _(p):
        pltpu.sync_copy(srcs.at[w, pl.ds(p * GCHUNK, GCHUNK)], sidx)
        pltpu.sync_copy(dsts.at[w, pl.ds(p * GCHUNK, GCHUNK)], didx)
        @pl.loop(0, GCHUNK, unroll=4)
        def _(j):
            pltpu.async_copy(g.at[sidx.at[j]], rows0, semg0).wait()
            pltpu.sync_copy(rows0, acc.at[didx.at[j]], add=True)

    plsc.subcore_barrier()
    _read_share(acc, rows0, out, cid, base)


BN = 400
BE = 2000


def _dinv_of(d_ref):
    deg = d_ref[0] + d_ref[1] + 1.0
    return lax.rsqrt(deg[:, :1])


def _tc_g1_body(x_ref, w_ref, d_ref, o_ref):
    h = jnp.dot(x_ref[...], w_ref[...], preferred_element_type=jnp.float32)
    o_ref[...] = h * _dinv_of(d_ref)


def _tc_g2_body(s_ref, g1_ref, d_ref, b1_ref, w2_ref, o_ref):
    dinv = _dinv_of(d_ref)
    node = (s_ref[0] + s_ref[1] + g1_ref[...]) * dinv + b1_ref[...]
    node = jnp.maximum(node, 0.0)
    h = jnp.dot(node, w2_ref[...], preferred_element_type=jnp.float32)
    o_ref[...] = h * dinv


def _tc_out_body(s_ref, g2_ref, d_ref, b2_ref, o_ref):
    dinv = _dinv_of(d_ref)
    o_ref[...] = (s_ref[0] + s_ref[1] + g2_ref[...]) * dinv + b2_ref[...]


def _tc_edge_body(a_ref, w1_ref, b1_ref, w2_ref, b2_ref, o_ref):
    hmid = jnp.dot(a_ref[...], w1_ref[...], preferred_element_type=jnp.float32)
    hmid = jnp.maximum(hmid + b1_ref[...], 0.0)
    o_ref[...] = (
        jnp.dot(hmid, w2_ref[...], preferred_element_type=jnp.float32)
        + b2_ref[...]
    )


def _row_spec(width):
    return pl.BlockSpec((BN, width), lambda i: (i, 0))


_D_SPEC = pl.BlockSpec((NC, BN, 16), lambda i: (0, i, 0))
_S_SPEC = pl.BlockSpec((NC, BN, H), lambda i: (0, i, 0))
_FULL = pl.BlockSpec((H, H), lambda i: (0, 0))
_BIAS = pl.BlockSpec((1, H), lambda i: (0, 0))

_tc_g1 = pl.pallas_call(
    _tc_g1_body,
    grid=(N // BN,),
    in_specs=[_row_spec(DF), _FULL, _D_SPEC],
    out_specs=_row_spec(H),
    out_shape=jax.ShapeDtypeStruct((N, H), jnp.float32),
)

_tc_g2 = pl.pallas_call(
    _tc_g2_body,
    grid=(N // BN,),
    in_specs=[_S_SPEC, _row_spec(H), _D_SPEC, _BIAS, _FULL],
    out_specs=_row_spec(H),
    out_shape=jax.ShapeDtypeStruct((N, H), jnp.float32),
)

_tc_out = pl.pallas_call(
    _tc_out_body,
    grid=(N // BN,),
    in_specs=[_S_SPEC, _row_spec(H), _D_SPEC, _BIAS],
    out_specs=_row_spec(H),
    out_shape=jax.ShapeDtypeStruct((N, H), jnp.float32),
)

_tc_edge = pl.pallas_call(
    _tc_edge_body,
    grid=(E // BE,),
    in_specs=[
        pl.BlockSpec((BE, DE), lambda i: (i, 0)),
        pl.BlockSpec((DE, H), lambda i: (0, 0)),
        _BIAS,
        _FULL,
        _BIAS,
    ],
    out_specs=pl.BlockSpec((BE, H), lambda i: (i, 0)),
    out_shape=jax.ShapeDtypeStruct((E, H), jnp.float32),
)


@functools.lru_cache(maxsize=None)
def _sc_kernels():
    mesh = plsc.VectorSubcoreMesh(
        core_axis_name="c", subcore_axis_name="s",
        num_cores=NC, num_subcores=NS,
    )
    sc_degree = pl.kernel(
        _sc_degree_body,
        out_type=jax.ShapeDtypeStruct((NC, NROWS, 16), jnp.float32),
        mesh=mesh,
        scratch_types=[
            pltpu.VMEM((NCHUNK, CHUNK), jnp.int32),
            pltpu.VMEM((CHUNK, 16), jnp.float32),
            pltpu.VMEM((CHUNK, 16), jnp.float32),
            pltpu.VMEM_SHARED((NROWS, 16), jnp.float32),
        ],
    )
    sc_scatter = pl.kernel(
        _sc_scatter_body,
        out_type=jax.ShapeDtypeStruct((NC, NROWS, H), jnp.float32),
        mesh=mesh,
        scratch_types=[
            pltpu.VMEM((GCHUNK, CHUNK), jnp.int32),
            pltpu.VMEM((GCHUNK, CHUNK), jnp.int32),
            pltpu.VMEM((CHUNK, H), jnp.float32),
            pltpu.VMEM((CHUNK, H), jnp.float32),
            pltpu.VMEM_SHARED((NROWS, H), jnp.float32),
            pltpu.SemaphoreType.DMA,
            pltpu.SemaphoreType.DMA,
        ],
    )
    return sc_degree, sc_scatter


def kernel(x, edge_index, edge_attr, W1, b1, W2, b2, We1, be1, We2, be2):
    _sc_degree, _sc_scatter = _sc_kernels()
    src = edge_index[0].astype(jnp.int32)
    dst = edge_index[1].astype(jnp.int32)
    srcs = jnp.pad(src.reshape(NW, EPW), ((0, 0), (0, EPAD)))
    srcs = srcs.reshape(NW, NCHUNK, CHUNK)
    dsts = jnp.pad(dst.reshape(NW, EPW), ((0, 0), (0, EPAD)),
                   constant_values=N)
    dsts = dsts.reshape(NW, NCHUNK, CHUNK)
    ones16 = jnp.ones((CHUNK, 16), jnp.float32)
    zer16 = jnp.zeros((CHUNK, 16), jnp.float32)
    zer128 = jnp.zeros((CHUNK, H), jnp.float32)

    edge_emb = _tc_edge(edge_attr, We1, be1.reshape(1, H), We2,
                        be2.reshape(1, H))

    d16 = _sc_degree(dsts, ones16, zer16)[:, :N]
    g1 = _tc_g1(x, W1, d16)
    s1 = _sc_scatter(g1, srcs, dsts, zer128)[:, :N]
    g2 = _tc_g2(s1, g1, d16, b1.reshape(1, H), W2)
    s2 = _sc_scatter(g2, srcs, dsts, zer128)[:, :N]
    node_emb = _tc_out(s2, g2, d16, b2.reshape(1, H))
    return (node_emb, edge_emb)


# speedup vs baseline: 1.3723x; 1.3723x over previous
import functools

import jax
import jax.numpy as jnp
from jax import lax
from jax.experimental import pallas as pl
from jax.experimental.pallas import tpu as pltpu
from jax.experimental.pallas import tpu_sc as plsc

N = 10000
E = 320000
DF = 128
DE = 16
H = 128

NC = 2
NS = 16
NW = NC * NS
EPW = E // NW
CHUNK = 128
NCHUNK = -(-EPW // CHUNK)
EPAD = NCHUNK * CHUNK - EPW
RPW = 320
NROWS = NW * RPW

def _zero_share(zbuf, acc, base):
    pltpu.sync_copy(zbuf, acc.at[pl.ds(base, CHUNK)])
    pltpu.sync_copy(zbuf, acc.at[pl.ds(base + CHUNK, CHUNK)])
    pltpu.sync_copy(zbuf.at[pl.ds(0, RPW - 2 * CHUNK)],
                    acc.at[pl.ds(base + 2 * CHUNK, RPW - 2 * CHUNK)])


def _read_share(acc, buf, out_hbm, cid, base):
    for lo, n in ((0, CHUNK), (CHUNK, CHUNK), (2 * CHUNK, RPW - 2 * CHUNK)):
        pltpu.sync_copy(acc.at[pl.ds(base + lo, n)], buf.at[pl.ds(0, n)])
        pltpu.sync_copy(buf.at[pl.ds(0, n)], out_hbm.at[cid, pl.ds(base + lo, n)])


def _sc_degree_body(dsts, ones, zeros, out, didx, obuf, zbuf, acc):
    cid = lax.axis_index("c")
    sid = lax.axis_index("s")
    w = cid * NS + sid
    pltpu.sync_copy(dsts.at[w], didx)
    pltpu.sync_copy(ones, obuf)
    pltpu.sync_copy(zeros, zbuf)
    base = sid * RPW
    _zero_share(zbuf, acc, base)
    plsc.subcore_barrier()

    @pl.loop(0, NCHUNK)
    def _(j):
        pltpu.sync_copy(obuf, acc.at[didx.at[j]], add=True)

    plsc.subcore_barrier()
    _read_share(acc, zbuf, out, cid, base)


def _sc_scatter_body(g, srcs, dsts, zeros, out, sidx, didx, rows, acc, sem0):
    cid = lax.axis_index("c")
    sid = lax.axis_index("s")
    w = cid * NS + sid
    base = sid * RPW
    pltpu.sync_copy(srcs.at[w], sidx)
    pltpu.sync_copy(dsts.at[w], didx)
    pltpu.sync_copy(zeros, rows)
    _zero_share(rows, acc, base)
    plsc.subcore_barrier()

    @pl.loop(0, NCHUNK)
    def _(j):
        pltpu.async_copy(g.at[sidx.at[j]], rows, sem0).wait()
        pltpu.sync_copy(rows, acc.at[didx.at[j]], add=True)

    plsc.subcore_barrier()
    _read_share(acc, rows, out, cid, base)


BN = 400
BE = 2000


def _dinv_of(d_ref):
    deg = d_ref[0] + d_ref[1] + 1.0
    return lax.rsqrt(deg[:, :1])


def _tc_g1_body(x_ref, w_ref, d_ref, o_ref):
    h = jnp.dot(x_ref[...], w_ref[...], preferred_element_type=jnp.float32)
    o_ref[...] = h * _dinv_of(d_ref)


def _tc_g2_body(s_ref, g1_ref, d_ref, b1_ref, w2_ref, o_ref):
    dinv = _dinv_of(d_ref)
    node = (s_ref[0] + s_ref[1] + g1_ref[...]) * dinv + b1_ref[...]
    node = jnp.maximum(node, 0.0)
    h = jnp.dot(node, w2_ref[...], preferred_element_type=jnp.float32)
    o_ref[...] = h * dinv


def _tc_out_body(s_ref, g2_ref, d_ref, b2_ref, o_ref):
    dinv = _dinv_of(d_ref)
    o_ref[...] = (s_ref[0] + s_ref[1] + g2_ref[...]) * dinv + b2_ref[...]


def _tc_edge_body(a_ref, w1_ref, b1_ref, w2_ref, b2_ref, o_ref):
    hmid = jnp.dot(a_ref[...], w1_ref[...], preferred_element_type=jnp.float32)
    hmid = jnp.maximum(hmid + b1_ref[...], 0.0)
    o_ref[...] = (
        jnp.dot(hmid, w2_ref[...], preferred_element_type=jnp.float32)
        + b2_ref[...]
    )


def _row_spec(width):
    return pl.BlockSpec((BN, width), lambda i: (i, 0))


_D_SPEC = pl.BlockSpec((NC, BN, 16), lambda i: (0, i, 0))
_S_SPEC = pl.BlockSpec((NC, BN, H), lambda i: (0, i, 0))
_FULL = pl.BlockSpec((H, H), lambda i: (0, 0))
_BIAS = pl.BlockSpec((1, H), lambda i: (0, 0))

_tc_g1 = pl.pallas_call(
    _tc_g1_body,
    grid=(N // BN,),
    in_specs=[_row_spec(DF), _FULL, _D_SPEC],
    out_specs=_row_spec(H),
    out_shape=jax.ShapeDtypeStruct((N, H), jnp.float32),
)

_tc_g2 = pl.pallas_call(
    _tc_g2_body,
    grid=(N // BN,),
    in_specs=[_S_SPEC, _row_spec(H), _D_SPEC, _BIAS, _FULL],
    out_specs=_row_spec(H),
    out_shape=jax.ShapeDtypeStruct((N, H), jnp.float32),
)

_tc_out = pl.pallas_call(
    _tc_out_body,
    grid=(N // BN,),
    in_specs=[_S_SPEC, _row_spec(H), _D_SPEC, _BIAS],
    out_specs=_row_spec(H),
    out_shape=jax.ShapeDtypeStruct((N, H), jnp.float32),
)

_tc_edge = pl.pallas_call(
    _tc_edge_body,
    grid=(E // BE,),
    in_specs=[
        pl.BlockSpec((BE, DE), lambda i: (i, 0)),
        pl.BlockSpec((DE, H), lambda i: (0, 0)),
        _BIAS,
        _FULL,
        _BIAS,
    ],
    out_specs=pl.BlockSpec((BE, H), lambda i: (i, 0)),
    out_shape=jax.ShapeDtypeStruct((E, H), jnp.float32),
)


@functools.lru_cache(maxsize=None)
def _sc_kernels():
    mesh = plsc.VectorSubcoreMesh(
        core_axis_name="c", subcore_axis_name="s",
        num_cores=NC, num_subcores=NS,
    )
    sc_degree = pl.kernel(
        _sc_degree_body,
        out_type=jax.ShapeDtypeStruct((NC, NROWS, 16), jnp.float32),
        mesh=mesh,
        scratch_types=[
            pltpu.VMEM((NCHUNK, CHUNK), jnp.int32),
            pltpu.VMEM((CHUNK, 16), jnp.float32),
            pltpu.VMEM((CHUNK, 16), jnp.float32),
            pltpu.VMEM_SHARED((NROWS, 16), jnp.float32),
        ],
    )
    sc_scatter = pl.kernel(
        _sc_scatter_body,
        out_type=jax.ShapeDtypeStruct((NC, NROWS, H), jnp.float32),
        mesh=mesh,
        scratch_types=[
            pltpu.VMEM((NCHUNK, CHUNK), jnp.int32),
            pltpu.VMEM((NCHUNK, CHUNK), jnp.int32),
            pltpu.VMEM((CHUNK, H), jnp.float32),
            pltpu.VMEM_SHARED((NROWS, H), jnp.float32),
            pltpu.SemaphoreType.DMA,
        ],
    )
    return sc_degree, sc_scatter


def kernel(x, edge_index, edge_attr, W1, b1, W2, b2, We1, be1, We2, be2):
    _sc_degree, _sc_scatter = _sc_kernels()
    src = edge_index[0].astype(jnp.int32)
    dst = edge_index[1].astype(jnp.int32)
    srcs = jnp.pad(src.reshape(NW, EPW), ((0, 0), (0, EPAD)))
    srcs = srcs.reshape(NW, NCHUNK, CHUNK)
    dsts = jnp.pad(dst.reshape(NW, EPW), ((0, 0), (0, EPAD)),
                   constant_values=N)
    dsts = dsts.reshape(NW, NCHUNK, CHUNK)
    ones16 = jnp.ones((CHUNK, 16), jnp.float32)
    zer16 = jnp.zeros((CHUNK, 16), jnp.float32)
    zer128 = jnp.zeros((CHUNK, H), jnp.float32)

    edge_emb = _tc_edge(edge_attr, We1, be1.reshape(1, H), We2,
                        be2.reshape(1, H))

    d16 = _sc_degree(dsts, ones16, zer16)[:, :N]
    g1 = _tc_g1(x, W1, d16)
    s1 = _sc_scatter(g1, srcs, dsts, zer128)[:, :N]
    g2 = _tc_g2(s1, g1, d16, b1.reshape(1, H), W2)
    s2 = _sc_scatter(g2, srcs, dsts, zer128)[:, :N]
    node_emb = _tc_out(s2, g2, d16, b2.reshape(1, H))
    return (node_emb, edge_emb)
